# baseline (speedup 1.0000x reference)
import functools
import math

import jax
import jax.numpy as jnp
from jax import lax
from jax.experimental import pallas as pl
from jax.experimental.pallas import tpu as pltpu
from jax.experimental.pallas import tpu_sc as plsc

N = 10000
E = 160000
C = 256
BINS = 256
NG = 128
VDIM = 32

BE = 1000
BN = 1000
NP = 10240
NSUB = 16
NCORE = 2
CH = 128

F32 = jnp.float32
BF16 = jnp.bfloat16


def _dotb(a, b):
    return jnp.dot(a.astype(F32), b.astype(F32), preferred_element_type=F32)


def _ln(x):
    m = jnp.mean(x, axis=-1, keepdims=True)
    v = jnp.mean((x - m) ** 2, axis=-1, keepdims=True)
    return (x - m) / jnp.sqrt(v + 1e-5)


def _silu(x):
    return x * jax.nn.sigmoid(x)


def _e_body(ea_ref, wr_ref, br_ref, out_ref):
    a = ea_ref[...]
    nrm = jnp.sqrt(jnp.sum(a * a, axis=1, keepdims=True))
    d = -0.75 / (nrm + 1e-9)
    cent = -4.0 + lax.broadcasted_iota(jnp.int32, (1, BINS), 1).astype(F32) * (
        4.0 / (BINS - 1))
    gamma = 1.0 / (4.0 / (BINS - 1))
    rbf = jnp.exp(-gamma * (d - cent) ** 2)
    z = _dotb(rbf, wr_ref[...]) + br_ref[...]
    out_ref[...] = jax.nn.softplus(z)


def _compute_e(ea8, Wr, br):
    return pl.pallas_call(
        _e_body,
        grid=(E // BE,),
        in_specs=[
            pl.BlockSpec((BE, 8), lambda i: (i, 0)),
            pl.BlockSpec((BINS, C), lambda i: (0, 0)),
            pl.BlockSpec((1, C), lambda i: (0, 0)),
        ],
        out_specs=pl.BlockSpec((BE, C), lambda i: (i, 0)),
        out_shape=jax.ShapeDtypeStruct((E, C), F32),
    )(ea8, Wr, br.reshape(1, C))


def _prep0_body(x_ref, wa_ref, ba_ref, wq_ref, bq_ref, wk_ref, bk_ref,
                wv_ref, bv_ref, h_ref, td_ref, ts_ref):
    h = jnp.dot(x_ref[...], wa_ref[...], preferred_element_type=F32) + ba_ref[...]
    q = jnp.dot(h, wq_ref[...], preferred_element_type=F32) + bq_ref[...]
    k = jnp.dot(h, wk_ref[...], preferred_element_type=F32) + bk_ref[...]
    v = jnp.dot(h, wv_ref[...], preferred_element_type=F32) + bv_ref[...]
    h_ref[...] = h
    td_ref[...] = jnp.concatenate([q, q * k, v], axis=1)
    ts_ref[...] = jnp.concatenate([k, v], axis=1)


def _prep0(xp, params):
    p0 = params['l0']
    return pl.pallas_call(
        _prep0_body,
        grid=(N // BN,),
        in_specs=[
            pl.BlockSpec((BN, 128), lambda i: (i, 0)),
            pl.BlockSpec((128, C), lambda i: (0, 0)),
            pl.BlockSpec((1, C), lambda i: (0, 0)),
            pl.BlockSpec((C, C), lambda i: (0, 0)),
            pl.BlockSpec((1, C), lambda i: (0, 0)),
            pl.BlockSpec((C, C), lambda i: (0, 0)),
            pl.BlockSpec((1, C), lambda i: (0, 0)),
            pl.BlockSpec((C, C), lambda i: (0, 0)),
            pl.BlockSpec((1, C), lambda i: (0, 0)),
        ],
        out_specs=[
            pl.BlockSpec((BN, C), lambda i: (i, 0)),
            pl.BlockSpec((BN, 3 * C), lambda i: (i, 0)),
            pl.BlockSpec((BN, 2 * C), lambda i: (i, 0)),
        ],
        out_shape=[
            jax.ShapeDtypeStruct((N, C), F32),
            jax.ShapeDtypeStruct((N, 3 * C), F32),
            jax.ShapeDtypeStruct((N, 2 * C), F32),
        ],
    )(xp, params['Wa_p'], params['ba'].reshape(1, C),
      p0['Wq'], p0['bq'].reshape(1, C), p0['Wk'], p0['bk'].reshape(1, C),
      p0['Wv'], p0['bv'].reshape(1, C))


def _edge_conv_body(gd_ref, gs_ref, e_ref, we_ref, be_ref,
                    wmu_ref, bmu_ref, wm_ref, bm_ref, m_ref):
    gd = gd_ref[...]
    gs = gs_ref[...]
    q_d = gd[:, :C]
    qk_d = gd[:, C:2 * C]
    v_d = gd[:, 2 * C:]
    k_s = gs[:, :C]
    v_s = gs[:, C:]
    e = e_ref[...]
    ee = _dotb(e, we_ref[...]) + be_ref[...]
    inv = 1.0 / math.sqrt(3 * C)
    alpha = jnp.concatenate([qk_d, q_d * k_s, q_d * ee], axis=1) * inv
    sig = jax.nn.sigmoid(_ln(alpha))
    wmu = wmu_ref[...]
    m_pre = (_dotb(v_d, wmu[:C]) + _dotb(v_s, wmu[C:2 * C])
             + _dotb(ee, wmu[2 * C:]) + bmu_ref[...])
    g = m_pre * sig
    m_ref[...] = _ln(_dotb(g, wm_ref[...]) + bm_ref[...])


def _edge_conv(gd, gs, e, p, off):
    ne = gd.shape[0]
    return pl.pallas_call(
        _edge_conv_body,
        grid=(ne // BE,),
        in_specs=[
            pl.BlockSpec((BE, 3 * C), lambda i: (i, 0)),
            pl.BlockSpec((BE, 2 * C), lambda i: (i, 0)),
            pl.BlockSpec((BE, C), lambda i: (i + off, 0)),
            pl.BlockSpec((C, C), lambda i: (0, 0)),
            pl.BlockSpec((1, C), lambda i: (0, 0)),
            pl.BlockSpec((3 * C, 3 * C), lambda i: (0, 0)),
            pl.BlockSpec((1, 3 * C), lambda i: (0, 0)),
            pl.BlockSpec((3 * C, C), lambda i: (0, 0)),
            pl.BlockSpec((1, C), lambda i: (0, 0)),
        ],
        out_specs=pl.BlockSpec((BE, C), lambda i: (i, 0)),
        out_shape=jax.ShapeDtypeStruct((ne, C), F32),
    )(gd, gs, e, p['We'], p['be'].reshape(1, C), p['Wmu'],
      p['bmu'].reshape(1, 3 * C), p['Wm'], p['bm'].reshape(1, C))


def _update0_body(h_ref, agg_ref, wc_ref, bc_ref, wn1_ref, h1_ref, hn1_ref):
    h1 = _silu(h_ref[...] + jnp.dot(agg_ref[...], wc_ref[...],
                                    preferred_element_type=F32) + bc_ref[...])
    h1_ref[...] = h1
    hn1_ref[...] = jnp.dot(h1, wn1_ref[...], preferred_element_type=F32)


def _update0(h, agg, params):
    return pl.pallas_call(
        _update0_body,
        grid=(N // BN,),
        in_specs=[
            pl.BlockSpec((BN, C), lambda i: (i, 0)),
            pl.BlockSpec((BN, C), lambda i: (i, 0)),
            pl.BlockSpec((C, C), lambda i: (0, 0)),
            pl.BlockSpec((1, C), lambda i: (0, 0)),
            pl.BlockSpec((C, C), lambda i: (0, 0)),
        ],
        out_specs=[
            pl.BlockSpec((BN, C), lambda i: (i, 0)),
            pl.BlockSpec((BN, C), lambda i: (i, 0)),
        ],
        out_shape=[
            jax.ShapeDtypeStruct((N, C), F32),
            jax.ShapeDtypeStruct((N, C), F32),
        ],
    )(h, agg, params['l0']['Wc'], params['l0']['bc'].reshape(1, C),
      params['eq']['Wn1'])


def _edge_equi_body(gn_ref, e_ref, ea_ref, wg_ref, wv1_ref, out_ref, mv_ref):
    gate = _silu(_dotb(e_ref[...], wg_ref[...]))
    m = gn_ref[...] * gate
    w = _dotb(m, wv1_ref[...])
    a = ea_ref[...]
    nrm = jnp.sqrt(jnp.sum(a * a, axis=1, keepdims=True))
    r = a / (nrm + 1e-9)
    out_ref[...] = m
    mv_ref[...] = jnp.concatenate(
        [w * r[:, 0:1], w * r[:, 1:2], w * r[:, 2:3],
         jnp.zeros((m.shape[0], 32), F32)], axis=1)


def _edge_equi(gn, e, ea8, params, off):
    eq = params['eq']
    ne = gn.shape[0]
    return pl.pallas_call(
        _edge_equi_body,
        grid=(ne // BE,),
        in_specs=[
            pl.BlockSpec((BE, C), lambda i: (i, 0)),
            pl.BlockSpec((BE, C), lambda i: (i + off, 0)),
            pl.BlockSpec((BE, 8), lambda i: (i + off, 0)),
            pl.BlockSpec((C, C), lambda i: (0, 0)),
            pl.BlockSpec((C, VDIM), lambda i: (0, 0)),
        ],
        out_specs=[
            pl.BlockSpec((BE, C), lambda i: (i, 0)),
            pl.BlockSpec((BE, 128), lambda i: (i, 0)),
        ],
        out_shape=[
            jax.ShapeDtypeStruct((ne, C), F32),
            jax.ShapeDtypeStruct((ne, 128), F32),
        ],
    )(gn, e, ea8, eq['Wg'], eq['Wv1'])


def _update1_body(h1_ref, agg_ref, aggv_ref, wn2_ref, wv2_ref, wq_ref, bq_ref,
                  wk_ref, bk_ref, wv_ref, bv_ref, h2_ref, td_ref, ts_ref):
    agg0 = agg_ref[...]
    av = aggv_ref[...]
    a = av[:, :96] + av[:, 128:224]
    inv = jnp.sqrt(a[:, :32] ** 2 + a[:, 32:64] ** 2 + a[:, 64:96] ** 2 + 1e-9)
    h2 = (h1_ref[...] + jnp.dot(agg0, wn2_ref[...], preferred_element_type=F32)
          + jnp.dot(inv, wv2_ref[...], preferred_element_type=F32))
    q = jnp.dot(h2, wq_ref[...], preferred_element_type=F32) + bq_ref[...]
    k = jnp.dot(h2, wk_ref[...], preferred_element_type=F32) + bk_ref[...]
    v = jnp.dot(h2, wv_ref[...], preferred_element_type=F32) + bv_ref[...]
    h2_ref[...] = h2
    td_ref[...] = jnp.concatenate([q, q * k, v], axis=1)
    ts_ref[...] = jnp.concatenate([k, v], axis=1)


def _update1(h1, agg_eq, aggv, params):
    eq = params['eq']
    p2 = params['l2']
    return pl.pallas_call(
        _update1_body,
        grid=(N // BN,),
        in_specs=[
            pl.BlockSpec((BN, C), lambda i: (i, 0)),
            pl.BlockSpec((BN, C), lambda i: (i, 0)),
            pl.BlockSpec((BN, C), lambda i: (i, 0)),
            pl.BlockSpec((C, C), lambda i: (0, 0)),
            pl.BlockSpec((VDIM, C), lambda i: (0, 0)),
            pl.BlockSpec((C, C), lambda i: (0, 0)),
            pl.BlockSpec((1, C), lambda i: (0, 0)),
            pl.BlockSpec((C, C), lambda i: (0, 0)),
            pl.BlockSpec((1, C), lambda i: (0, 0)),
            pl.BlockSpec((C, C), lambda i: (0, 0)),
            pl.BlockSpec((1, C), lambda i: (0, 0)),
        ],
        out_specs=[
            pl.BlockSpec((BN, C), lambda i: (i, 0)),
            pl.BlockSpec((BN, 3 * C), lambda i: (i, 0)),
            pl.BlockSpec((BN, 2 * C), lambda i: (i, 0)),
        ],
        out_shape=[
            jax.ShapeDtypeStruct((N, C), F32),
            jax.ShapeDtypeStruct((N, 3 * C), F32),
            jax.ShapeDtypeStruct((N, 2 * C), F32),
        ],
    )(h1, agg_eq, aggv, eq['Wn2'], eq['Wv2'],
      p2['Wq'], p2['bq'].reshape(1, C), p2['Wk'], p2['bk'].reshape(1, C),
      p2['Wv'], p2['bv'].reshape(1, C))


def _final_body(h2_ref, agg_ref, wc_ref, bc_ref, b_ref, wf_ref, bf_ref,
                wo_ref, bo_ref, sums_ref, cnt_ref, out_ref):
    i = pl.program_id(0)

    @pl.when(i == 0)
    def _():
        sums_ref[...] = jnp.zeros_like(sums_ref)
        cnt_ref[...] = jnp.zeros_like(cnt_ref)
        out_ref[...] = jnp.zeros_like(out_ref)

    h3 = _silu(h2_ref[...] + jnp.dot(agg_ref[...], wc_ref[...],
                                     preferred_element_type=F32) + bc_ref[...])
    brow = b_ref[0]
    gids = lax.broadcasted_iota(jnp.int32, (NG, BN), 0).astype(F32)
    onehot_t = (gids == brow).astype(F32)
    sums_ref[...] += jnp.dot(onehot_t, h3, preferred_element_type=F32)
    cnt_ref[...] += jnp.sum(onehot_t, axis=1, keepdims=True)

    @pl.when(i == pl.num_programs(0) - 1)
    def _():
        mean = sums_ref[...] / jnp.maximum(cnt_ref[...], 1.0)
        feat = _silu(jnp.dot(mean, wf_ref[...],
                             preferred_element_type=F32) + bf_ref[...])
        out_ref[...] = jnp.dot(feat, wo_ref[...],
                               preferred_element_type=F32) + bo_ref[...]


def _final(h2, agg2, batch3, params):
    p2 = params['l2']
    sums, cnt, out = pl.pallas_call(
        _final_body,
        grid=(N // BN,),
        in_specs=[
            pl.BlockSpec((BN, C), lambda i: (i, 0)),
            pl.BlockSpec((BN, C), lambda i: (i, 0)),
            pl.BlockSpec((C, C), lambda i: (0, 0)),
            pl.BlockSpec((1, C), lambda i: (0, 0)),
            pl.BlockSpec((1, 1, BN), lambda i: (i, 0, 0)),
            pl.BlockSpec((C, C), lambda i: (0, 0)),
            pl.BlockSpec((1, C), lambda i: (0, 0)),
            pl.BlockSpec((C, 1), lambda i: (0, 0)),
            pl.BlockSpec((1, 1), lambda i: (0, 0)),
        ],
        out_specs=[
            pl.BlockSpec((NG, C), lambda i: (0, 0)),
            pl.BlockSpec((NG, 1), lambda i: (0, 0)),
            pl.BlockSpec((NG, 1), lambda i: (0, 0)),
        ],
        out_shape=[
            jax.ShapeDtypeStruct((NG, C), F32),
            jax.ShapeDtypeStruct((NG, 1), F32),
            jax.ShapeDtypeStruct((NG, 1), F32),
        ],
    )(h2, agg2, p2['Wc'], p2['bc'].reshape(1, C), batch3,
      params['Wf'], params['bf'].reshape(1, C), params['Wo'],
      params['bo'].reshape(1, 1))
    return out


_NCHUNK = E // CH


def _mesh():
    return plsc.VectorSubcoreMesh(core_axis_name="c", subcore_axis_name="s")


_NPAIR = _NCHUNK // 64
_NTAIL = _NCHUNK - _NPAIR * 64


def _as_i32(t):
    return lax.bitcast_convert_type(
        t.reshape(t.shape[0], t.shape[1] // 2, 2), jnp.int32)


def _as_bf16(o, W):
    return lax.bitcast_convert_type(o, BF16).reshape(o.shape[0], W)


def _sc_gather(tab, idx):
    W = tab.shape[1]
    ne = idx.shape[0]
    nchunk = ne // CH

    @functools.partial(
        pl.kernel,
        mesh=_mesh(),
        out_type=jax.ShapeDtypeStruct((ne, W), F32),
        scratch_types=[
            pltpu.VMEM((CH,), jnp.int32),
            pltpu.VMEM((CH, W), F32),
            pltpu.SemaphoreType.DMA,
        ],
    )
    def gk(t_hbm, idx_hbm, o_hbm, idx_v, rows_v, sem):
        wid = lax.axis_index("s") * NCORE + lax.axis_index("c")

        @pl.loop(wid, nchunk, step=NCORE * NSUB)
        def _(j):
            base = j * CH
            pltpu.sync_copy(idx_hbm.at[pl.ds(base, CH)], idx_v)
            pltpu.async_copy(t_hbm.at[idx_v], rows_v, sem).wait()
            pltpu.sync_copy(rows_v, o_hbm.at[pl.ds(base, CH)])

    return gk(tab, idx)


def _sc_gather_ds(td, ts, idx_d, idx_s):
    Wd = td.shape[1]
    Ws = ts.shape[1]
    CH2 = 80
    ne = idx_d.shape[0]
    nchunk = ne // CH2

    @functools.partial(
        pl.kernel,
        mesh=_mesh(),
        out_type=[jax.ShapeDtypeStruct((ne, Wd), F32),
                  jax.ShapeDtypeStruct((ne, Ws), F32)],
        scratch_types=[
            pltpu.VMEM((CH2,), jnp.int32),
            pltpu.VMEM((CH2,), jnp.int32),
            pltpu.VMEM((CH2, Wd), F32),
            pltpu.VMEM((CH2, Ws), F32),
            pltpu.SemaphoreType.DMA,
            pltpu.SemaphoreType.DMA,
        ],
    )
    def gk(td_hbm, ts_hbm, id_hbm, is_hbm, od_hbm, os_hbm,
           ia, ib, rd, rs, sd, ss):
        wid = lax.axis_index("s") * NCORE + lax.axis_index("c")

        @pl.loop(wid, nchunk, step=NCORE * NSUB)
        def _(j):
            base = j * CH2
            pltpu.sync_copy(id_hbm.at[pl.ds(base, CH2)], ia)
            pltpu.sync_copy(is_hbm.at[pl.ds(base, CH2)], ib)
            gd_cp = pltpu.async_copy(td_hbm.at[ia], rd, sd)
            gs_cp = pltpu.async_copy(ts_hbm.at[ib], rs, ss)
            gd_cp.wait()
            pltpu.sync_copy(rd, od_hbm.at[pl.ds(base, CH2)])
            gs_cp.wait()
            pltpu.sync_copy(rs, os_hbm.at[pl.ds(base, CH2)])

    return gk(td, ts, idx_d, idx_s)


def _sc_scatter_add(vals, idx, init, D):
    Dh = D // 2
    rps = NP // NSUB
    ne = idx.shape[0]
    nchunk = ne // CH

    @functools.partial(
        pl.kernel,
        mesh=_mesh(),
        out_type=jax.ShapeDtypeStruct((NP, D), F32),
        scratch_types=[
            pltpu.VMEM((CH,), jnp.int32),
            pltpu.VMEM((CH, Dh), F32),
            pltpu.VMEM_SHARED((NP, Dh), F32),
            pltpu.SemaphoreType.DMA,
        ],
    )
    def sk(vals_hbm, idx_hbm, init_hbm, out_hbm, idx_v, vals_v, acc, sem):
        c = lax.axis_index("c")
        s = lax.axis_index("s")
        pltpu.sync_copy(init_hbm.at[pl.ds(s * rps, rps), pl.ds(c * Dh, Dh)],
                        acc.at[pl.ds(s * rps, rps)])
        plsc.subcore_barrier()

        @pl.loop(s, nchunk, step=NSUB)
        def _(j):
            base = j * CH
            pltpu.sync_copy(idx_hbm.at[pl.ds(base, CH)], idx_v)
            pltpu.sync_copy(vals_hbm.at[pl.ds(base, CH), pl.ds(c * Dh, Dh)],
                            vals_v)
            pltpu.sync_copy(vals_v, acc.at[idx_v], add=True)

        plsc.subcore_barrier()
        pltpu.sync_copy(acc.at[pl.ds(s * rps, rps)],
                        out_hbm.at[pl.ds(s * rps, rps), pl.ds(c * Dh, Dh)])

    return sk(vals, idx, init)


def _sc_scatter_add_esplit(vals, idx, init):
    Dh = 128
    rps = NP // NSUB
    ne = idx.shape[0]
    nchunk = ne // CH
    half = nchunk // 2

    @functools.partial(
        pl.kernel,
        mesh=_mesh(),
        out_type=jax.ShapeDtypeStruct((NP, 2 * Dh), F32),
        scratch_types=[
            pltpu.VMEM((CH,), jnp.int32),
            pltpu.VMEM((CH, Dh), F32),
            pltpu.VMEM_SHARED((NP, Dh), F32),
            pltpu.SemaphoreType.DMA,
        ],
    )
    def sk(vals_hbm, idx_hbm, init_hbm, out_hbm, idx_v, vals_v, acc, sem):
        c = lax.axis_index("c")
        s = lax.axis_index("s")
        pltpu.sync_copy(init_hbm.at[pl.ds(s * rps, rps), pl.ds(c * Dh, Dh)],
                        acc.at[pl.ds(s * rps, rps)])
        plsc.subcore_barrier()
        hi = jnp.where(c == 0, half, nchunk)

        @pl.loop(c * half + s, hi, step=NSUB)
        def _(j):
            base = j * CH
            pltpu.sync_copy(idx_hbm.at[pl.ds(base, CH)], idx_v)
            pltpu.sync_copy(vals_hbm.at[pl.ds(base, CH)], vals_v)
            pltpu.sync_copy(vals_v, acc.at[idx_v], add=True)

        plsc.subcore_barrier()
        pltpu.sync_copy(acc.at[pl.ds(s * rps, rps)],
                        out_hbm.at[pl.ds(s * rps, rps), pl.ds(c * Dh, Dh)])

    return sk(vals, idx, init)


def kernel(x, edge_index, edge_attr, batch, params):
    src = edge_index[0]
    dst = edge_index[1]
    ea8 = jnp.pad(edge_attr, ((0, 0), (0, 5)))
    xp = jnp.pad(x, ((0, 0), (0, 128 - x.shape[1])))
    pp = dict(params)
    pp['Wa_p'] = jnp.pad(params['Wa'], ((0, 128 - params['Wa'].shape[0]), (0, 0)))
    batch3 = batch.astype(F32).reshape(N // BN, 1, BN)
    znp = jnp.zeros((NP, 2 * 128), F32)
    E2 = E // 2
    HOFF = E2 // BE
    dst_a, dst_b = dst[:E2], dst[E2:]
    src_a, src_b = src[:E2], src[E2:]

    e = _compute_e(ea8, params['Wr'], params['br'])
    h0, td0, ts0 = _prep0(xp, pp)

    gd0a, gs0a = _sc_gather_ds(td0, ts0, dst_a, src_a)
    gd0b, gs0b = _sc_gather_ds(td0, ts0, dst_b, src_b)
    m0a = _edge_conv(gd0a, gs0a, e, params['l0'], 0)
    m0b = _edge_conv(gd0b, gs0b, e, params['l0'], HOFF)
    s0a = _sc_scatter_add(m0a, dst_a, znp, C)
    agg0 = _sc_scatter_add(m0b, dst_b, s0a, C)[:N]

    h1, hn1 = _update0(h0, agg0, params)
    gna = _sc_gather(hn1, src_a)
    gnb = _sc_gather(hn1, src_b)
    ma, mva = _edge_equi(gna, e, ea8, params, 0)
    mb, mvb = _edge_equi(gnb, e, ea8, params, HOFF)
    sea = _sc_scatter_add(ma, dst_a, znp, C)
    agg_eq = _sc_scatter_add(mb, dst_b, sea, C)[:N]
    va = _sc_scatter_add_esplit(mva, dst_a, znp)
    aggv = _sc_scatter_add_esplit(mvb, dst_b, va)[:N]

    h2, td2, ts2 = _update1(h1, agg_eq, aggv, params)
    gd2a, gs2a = _sc_gather_ds(td2, ts2, dst_a, src_a)
    gd2b, gs2b = _sc_gather_ds(td2, ts2, dst_b, src_b)
    m2a = _edge_conv(gd2a, gs2a, e, params['l2'], 0)
    m2b = _edge_conv(gd2b, gs2b, e, params['l2'], HOFF)
    s2a = _sc_scatter_add(m2a, dst_a, znp, C)
    agg2 = _sc_scatter_add(m2b, dst_b, s2a, C)[:N]

    out = _final(h2, agg2, batch3, params)
    return out.reshape(NG)


# speedup vs baseline: 1.0047x; 1.0047x over previous
import functools
import math

import jax
import jax.numpy as jnp
from jax import lax
from jax.experimental import pallas as pl
from jax.experimental.pallas import tpu as pltpu
from jax.experimental.pallas import tpu_sc as plsc

N = 10000
E = 160000
C = 256
BINS = 256
NG = 128
VDIM = 32

BE = 1000
BN = 1000
NP = 10240
NSUB = 16
NCORE = 2
CH = 128

F32 = jnp.float32


def _dotb(a, b):
    return jnp.dot(a, b, preferred_element_type=F32)


def _ln(x):
    m = jnp.mean(x, axis=-1, keepdims=True)
    v = jnp.mean((x - m) ** 2, axis=-1, keepdims=True)
    return (x - m) / jnp.sqrt(v + 1e-5)


def _silu(x):
    return x * jax.nn.sigmoid(x)


def _e_body(ea_ref, wr_ref, br_ref, out_ref):
    a = ea_ref[...]
    nrm = jnp.sqrt(jnp.sum(a * a, axis=1, keepdims=True))
    d = -0.75 / (nrm + 1e-9)
    cent = -4.0 + lax.broadcasted_iota(jnp.int32, (1, BINS), 1).astype(F32) * (
        4.0 / (BINS - 1))
    gamma = 1.0 / (4.0 / (BINS - 1))
    rbf = jnp.exp(-gamma * (d - cent) ** 2)
    z = _dotb(rbf, wr_ref[...]) + br_ref[...]
    out_ref[...] = jax.nn.softplus(z)


def _compute_e(ea8, Wr, br):
    return pl.pallas_call(
        _e_body,
        grid=(E // BE,),
        in_specs=[
            pl.BlockSpec((BE, 8), lambda i: (i, 0)),
            pl.BlockSpec((BINS, C), lambda i: (0, 0)),
            pl.BlockSpec((1, C), lambda i: (0, 0)),
        ],
        out_specs=pl.BlockSpec((BE, C), lambda i: (i, 0)),
        out_shape=jax.ShapeDtypeStruct((E, C), F32),
    )(ea8, Wr, br.reshape(1, C))


def _prep0_body(x_ref, wa_ref, ba_ref, wq_ref, bq_ref, wk_ref, bk_ref,
                wv_ref, bv_ref, h_ref, td_ref, ts_ref):
    h = jnp.dot(x_ref[...], wa_ref[...], preferred_element_type=F32) + ba_ref[...]
    q = jnp.dot(h, wq_ref[...], preferred_element_type=F32) + bq_ref[...]
    k = jnp.dot(h, wk_ref[...], preferred_element_type=F32) + bk_ref[...]
    v = jnp.dot(h, wv_ref[...], preferred_element_type=F32) + bv_ref[...]
    h_ref[...] = h
    td_ref[...] = jnp.concatenate([q, q * k, v], axis=1)
    ts_ref[...] = jnp.concatenate([k, v], axis=1)


def _prep0(xp, params):
    p0 = params['l0']
    return pl.pallas_call(
        _prep0_body,
        grid=(N // BN,),
        in_specs=[
            pl.BlockSpec((BN, 128), lambda i: (i, 0)),
            pl.BlockSpec((128, C), lambda i: (0, 0)),
            pl.BlockSpec((1, C), lambda i: (0, 0)),
            pl.BlockSpec((C, C), lambda i: (0, 0)),
            pl.BlockSpec((1, C), lambda i: (0, 0)),
            pl.BlockSpec((C, C), lambda i: (0, 0)),
            pl.BlockSpec((1, C), lambda i: (0, 0)),
            pl.BlockSpec((C, C), lambda i: (0, 0)),
            pl.BlockSpec((1, C), lambda i: (0, 0)),
        ],
        out_specs=[
            pl.BlockSpec((BN, C), lambda i: (i, 0)),
            pl.BlockSpec((BN, 3 * C), lambda i: (i, 0)),
            pl.BlockSpec((BN, 2 * C), lambda i: (i, 0)),
        ],
        out_shape=[
            jax.ShapeDtypeStruct((N, C), F32),
            jax.ShapeDtypeStruct((N, 3 * C), F32),
            jax.ShapeDtypeStruct((N, 2 * C), F32),
        ],
    )(xp, params['Wa_p'], params['ba'].reshape(1, C),
      p0['Wq'], p0['bq'].reshape(1, C), p0['Wk'], p0['bk'].reshape(1, C),
      p0['Wv'], p0['bv'].reshape(1, C))


def _edge_conv_body(gd_ref, gs_ref, e_ref, we_ref, be_ref,
                    wmu_ref, bmu_ref, wm_ref, bm_ref, m_ref):
    gd = gd_ref[...]
    gs = gs_ref[...]
    q_d = gd[:, :C]
    qk_d = gd[:, C:2 * C]
    v_d = gd[:, 2 * C:]
    k_s = gs[:, :C]
    v_s = gs[:, C:]
    e = e_ref[...]
    ee = _dotb(e, we_ref[...]) + be_ref[...]
    inv = 1.0 / math.sqrt(3 * C)
    alpha = jnp.concatenate([qk_d, q_d * k_s, q_d * ee], axis=1) * inv
    sig = jax.nn.sigmoid(_ln(alpha))
    wmu = wmu_ref[...]
    m_pre = (_dotb(v_d, wmu[:C]) + _dotb(v_s, wmu[C:2 * C])
             + _dotb(ee, wmu[2 * C:]) + bmu_ref[...])
    g = m_pre * sig
    m_ref[...] = _ln(_dotb(g, wm_ref[...]) + bm_ref[...])


def _edge_conv(gd, gs, e, p, off):
    ne = gd.shape[0]
    return pl.pallas_call(
        _edge_conv_body,
        grid=(ne // BE,),
        in_specs=[
            pl.BlockSpec((BE, 3 * C), lambda i: (i, 0)),
            pl.BlockSpec((BE, 2 * C), lambda i: (i, 0)),
            pl.BlockSpec((BE, C), lambda i: (i + off, 0)),
            pl.BlockSpec((C, C), lambda i: (0, 0)),
            pl.BlockSpec((1, C), lambda i: (0, 0)),
            pl.BlockSpec((3 * C, 3 * C), lambda i: (0, 0)),
            pl.BlockSpec((1, 3 * C), lambda i: (0, 0)),
            pl.BlockSpec((3 * C, C), lambda i: (0, 0)),
            pl.BlockSpec((1, C), lambda i: (0, 0)),
        ],
        out_specs=pl.BlockSpec((BE, C), lambda i: (i, 0)),
        out_shape=jax.ShapeDtypeStruct((ne, C), F32),
    )(gd, gs, e, p['We'], p['be'].reshape(1, C), p['Wmu'],
      p['bmu'].reshape(1, 3 * C), p['Wm'], p['bm'].reshape(1, C))


def _update0_body(h_ref, agg_ref, wc_ref, bc_ref, wn1_ref, h1_ref, hn1_ref):
    h1 = _silu(h_ref[...] + jnp.dot(agg_ref[...], wc_ref[...],
                                    preferred_element_type=F32) + bc_ref[...])
    h1_ref[...] = h1
    hn1_ref[...] = jnp.dot(h1, wn1_ref[...], preferred_element_type=F32)


def _update0(h, agg, params):
    return pl.pallas_call(
        _update0_body,
        grid=(N // BN,),
        in_specs=[
            pl.BlockSpec((BN, C), lambda i: (i, 0)),
            pl.BlockSpec((BN, C), lambda i: (i, 0)),
            pl.BlockSpec((C, C), lambda i: (0, 0)),
            pl.BlockSpec((1, C), lambda i: (0, 0)),
            pl.BlockSpec((C, C), lambda i: (0, 0)),
        ],
        out_specs=[
            pl.BlockSpec((BN, C), lambda i: (i, 0)),
            pl.BlockSpec((BN, C), lambda i: (i, 0)),
        ],
        out_shape=[
            jax.ShapeDtypeStruct((N, C), F32),
            jax.ShapeDtypeStruct((N, C), F32),
        ],
    )(h, agg, params['l0']['Wc'], params['l0']['bc'].reshape(1, C),
      params['eq']['Wn1'])


def _edge_equi_body(gn_ref, e_ref, ea_ref, wg_ref, wv1_ref, out_ref, mv_ref):
    gate = _silu(_dotb(e_ref[...], wg_ref[...]))
    m = gn_ref[...] * gate
    w = _dotb(m, wv1_ref[...])
    a = ea_ref[...]
    nrm = jnp.sqrt(jnp.sum(a * a, axis=1, keepdims=True))
    r = a / (nrm + 1e-9)
    out_ref[...] = m
    mv_ref[...] = jnp.concatenate(
        [w * r[:, 0:1], w * r[:, 1:2], w * r[:, 2:3],
         jnp.zeros((m.shape[0], 32), F32)], axis=1)


def _edge_equi(gn, e, ea8, params, off):
    eq = params['eq']
    ne = gn.shape[0]
    return pl.pallas_call(
        _edge_equi_body,
        grid=(ne // BE,),
        in_specs=[
            pl.BlockSpec((BE, C), lambda i: (i, 0)),
            pl.BlockSpec((BE, C), lambda i: (i + off, 0)),
            pl.BlockSpec((BE, 8), lambda i: (i + off, 0)),
            pl.BlockSpec((C, C), lambda i: (0, 0)),
            pl.BlockSpec((C, VDIM), lambda i: (0, 0)),
        ],
        out_specs=[
            pl.BlockSpec((BE, C), lambda i: (i, 0)),
            pl.BlockSpec((BE, 128), lambda i: (i, 0)),
        ],
        out_shape=[
            jax.ShapeDtypeStruct((ne, C), F32),
            jax.ShapeDtypeStruct((ne, 128), F32),
        ],
    )(gn, e, ea8, eq['Wg'], eq['Wv1'])


def _update1_body(h1_ref, agg_ref, aggv_ref, wn2_ref, wv2_ref, wq_ref, bq_ref,
                  wk_ref, bk_ref, wv_ref, bv_ref, h2_ref, td_ref, ts_ref):
    agg0 = agg_ref[...]
    av = aggv_ref[...]
    a = av[:, :96] + av[:, 128:224]
    inv = jnp.sqrt(a[:, :32] ** 2 + a[:, 32:64] ** 2 + a[:, 64:96] ** 2 + 1e-9)
    h2 = (h1_ref[...] + jnp.dot(agg0, wn2_ref[...], preferred_element_type=F32)
          + jnp.dot(inv, wv2_ref[...], preferred_element_type=F32))
    q = jnp.dot(h2, wq_ref[...], preferred_element_type=F32) + bq_ref[...]
    k = jnp.dot(h2, wk_ref[...], preferred_element_type=F32) + bk_ref[...]
    v = jnp.dot(h2, wv_ref[...], preferred_element_type=F32) + bv_ref[...]
    h2_ref[...] = h2
    td_ref[...] = jnp.concatenate([q, q * k, v], axis=1)
    ts_ref[...] = jnp.concatenate([k, v], axis=1)


def _update1(h1, agg_eq, aggv, params):
    eq = params['eq']
    p2 = params['l2']
    return pl.pallas_call(
        _update1_body,
        grid=(N // BN,),
        in_specs=[
            pl.BlockSpec((BN, C), lambda i: (i, 0)),
            pl.BlockSpec((BN, C), lambda i: (i, 0)),
            pl.BlockSpec((BN, C), lambda i: (i, 0)),
            pl.BlockSpec((C, C), lambda i: (0, 0)),
            pl.BlockSpec((VDIM, C), lambda i: (0, 0)),
            pl.BlockSpec((C, C), lambda i: (0, 0)),
            pl.BlockSpec((1, C), lambda i: (0, 0)),
            pl.BlockSpec((C, C), lambda i: (0, 0)),
            pl.BlockSpec((1, C), lambda i: (0, 0)),
            pl.BlockSpec((C, C), lambda i: (0, 0)),
            pl.BlockSpec((1, C), lambda i: (0, 0)),
        ],
        out_specs=[
            pl.BlockSpec((BN, C), lambda i: (i, 0)),
            pl.BlockSpec((BN, 3 * C), lambda i: (i, 0)),
            pl.BlockSpec((BN, 2 * C), lambda i: (i, 0)),
        ],
        out_shape=[
            jax.ShapeDtypeStruct((N, C), F32),
            jax.ShapeDtypeStruct((N, 3 * C), F32),
            jax.ShapeDtypeStruct((N, 2 * C), F32),
        ],
    )(h1, agg_eq, aggv, eq['Wn2'], eq['Wv2'],
      p2['Wq'], p2['bq'].reshape(1, C), p2['Wk'], p2['bk'].reshape(1, C),
      p2['Wv'], p2['bv'].reshape(1, C))


def _final_body(h2_ref, agg_ref, wc_ref, bc_ref, b_ref, wf_ref, bf_ref,
                wo_ref, bo_ref, sums_ref, cnt_ref, out_ref):
    i = pl.program_id(0)

    @pl.when(i == 0)
    def _():
        sums_ref[...] = jnp.zeros_like(sums_ref)
        cnt_ref[...] = jnp.zeros_like(cnt_ref)
        out_ref[...] = jnp.zeros_like(out_ref)

    h3 = _silu(h2_ref[...] + jnp.dot(agg_ref[...], wc_ref[...],
                                     preferred_element_type=F32) + bc_ref[...])
    brow = b_ref[0]
    gids = lax.broadcasted_iota(jnp.int32, (NG, BN), 0).astype(F32)
    onehot_t = (gids == brow).astype(F32)
    sums_ref[...] += jnp.dot(onehot_t, h3, preferred_element_type=F32)
    cnt_ref[...] += jnp.sum(onehot_t, axis=1, keepdims=True)

    @pl.when(i == pl.num_programs(0) - 1)
    def _():
        mean = sums_ref[...] / jnp.maximum(cnt_ref[...], 1.0)
        feat = _silu(jnp.dot(mean, wf_ref[...],
                             preferred_element_type=F32) + bf_ref[...])
        out_ref[...] = jnp.dot(feat, wo_ref[...],
                               preferred_element_type=F32) + bo_ref[...]


def _final(h2, agg2, batch3, params):
    p2 = params['l2']
    sums, cnt, out = pl.pallas_call(
        _final_body,
        grid=(N // BN,),
        in_specs=[
            pl.BlockSpec((BN, C), lambda i: (i, 0)),
            pl.BlockSpec((BN, C), lambda i: (i, 0)),
            pl.BlockSpec((C, C), lambda i: (0, 0)),
            pl.BlockSpec((1, C), lambda i: (0, 0)),
            pl.BlockSpec((1, 1, BN), lambda i: (i, 0, 0)),
            pl.BlockSpec((C, C), lambda i: (0, 0)),
            pl.BlockSpec((1, C), lambda i: (0, 0)),
            pl.BlockSpec((C, 1), lambda i: (0, 0)),
            pl.BlockSpec((1, 1), lambda i: (0, 0)),
        ],
        out_specs=[
            pl.BlockSpec((NG, C), lambda i: (0, 0)),
            pl.BlockSpec((NG, 1), lambda i: (0, 0)),
            pl.BlockSpec((NG, 1), lambda i: (0, 0)),
        ],
        out_shape=[
            jax.ShapeDtypeStruct((NG, C), F32),
            jax.ShapeDtypeStruct((NG, 1), F32),
            jax.ShapeDtypeStruct((NG, 1), F32),
        ],
    )(h2, agg2, p2['Wc'], p2['bc'].reshape(1, C), batch3,
      params['Wf'], params['bf'].reshape(1, C), params['Wo'],
      params['bo'].reshape(1, 1))
    return out


_NCHUNK = E // CH


def _mesh():
    return plsc.VectorSubcoreMesh(core_axis_name="c", subcore_axis_name="s")


def _sc_gather(tab, idx):
    W = tab.shape[1]
    ne = idx.shape[0]
    nchunk = ne // CH

    @functools.partial(
        pl.kernel,
        mesh=_mesh(),
        out_type=jax.ShapeDtypeStruct((ne, W), F32),
        scratch_types=[
            pltpu.VMEM((CH,), jnp.int32),
            pltpu.VMEM((CH, W), F32),
            pltpu.SemaphoreType.DMA,
        ],
    )
    def gk(t_hbm, idx_hbm, o_hbm, idx_v, rows_v, sem):
        wid = lax.axis_index("s") * NCORE + lax.axis_index("c")

        @pl.loop(wid, nchunk, step=NCORE * NSUB)
        def _(j):
            base = j * CH
            pltpu.sync_copy(idx_hbm.at[pl.ds(base, CH)], idx_v)
            pltpu.async_copy(t_hbm.at[idx_v], rows_v, sem).wait()
            pltpu.sync_copy(rows_v, o_hbm.at[pl.ds(base, CH)])

    return gk(tab, idx)


def _sc_scatter_add(vals, idx, init, D):
    Dh = D // 2
    rps = NP // NSUB
    ne = idx.shape[0]
    nchunk = ne // CH

    @functools.partial(
        pl.kernel,
        mesh=_mesh(),
        out_type=jax.ShapeDtypeStruct((NP, D), F32),
        scratch_types=[
            pltpu.VMEM((CH,), jnp.int32),
            pltpu.VMEM((CH, Dh), F32),
            pltpu.VMEM_SHARED((NP, Dh), F32),
            pltpu.SemaphoreType.DMA,
        ],
    )
    def sk(vals_hbm, idx_hbm, init_hbm, out_hbm, idx_v, vals_v, acc, sem):
        c = lax.axis_index("c")
        s = lax.axis_index("s")
        pltpu.sync_copy(init_hbm.at[pl.ds(s * rps, rps), pl.ds(c * Dh, Dh)],
                        acc.at[pl.ds(s * rps, rps)])
        plsc.subcore_barrier()

        @pl.loop(s, nchunk, step=NSUB)
        def _(j):
            base = j * CH
            pltpu.sync_copy(idx_hbm.at[pl.ds(base, CH)], idx_v)
            pltpu.sync_copy(vals_hbm.at[pl.ds(base, CH), pl.ds(c * Dh, Dh)],
                            vals_v)
            pltpu.sync_copy(vals_v, acc.at[idx_v], add=True)

        plsc.subcore_barrier()
        pltpu.sync_copy(acc.at[pl.ds(s * rps, rps)],
                        out_hbm.at[pl.ds(s * rps, rps), pl.ds(c * Dh, Dh)])

    return sk(vals, idx, init)


def _sc_scatter_add_esplit(vals, idx, init):
    Dh = 128
    rps = NP // NSUB
    ne = idx.shape[0]
    nchunk = ne // CH
    half = nchunk // 2

    @functools.partial(
        pl.kernel,
        mesh=_mesh(),
        out_type=jax.ShapeDtypeStruct((NP, 2 * Dh), F32),
        scratch_types=[
            pltpu.VMEM((CH,), jnp.int32),
            pltpu.VMEM((CH, Dh), F32),
            pltpu.VMEM_SHARED((NP, Dh), F32),
            pltpu.SemaphoreType.DMA,
        ],
    )
    def sk(vals_hbm, idx_hbm, init_hbm, out_hbm, idx_v, vals_v, acc, sem):
        c = lax.axis_index("c")
        s = lax.axis_index("s")
        pltpu.sync_copy(init_hbm.at[pl.ds(s * rps, rps), pl.ds(c * Dh, Dh)],
                        acc.at[pl.ds(s * rps, rps)])
        plsc.subcore_barrier()
        hi = jnp.where(c == 0, half, nchunk)

        @pl.loop(c * half + s, hi, step=NSUB)
        def _(j):
            base = j * CH
            pltpu.sync_copy(idx_hbm.at[pl.ds(base, CH)], idx_v)
            pltpu.sync_copy(vals_hbm.at[pl.ds(base, CH)], vals_v)
            pltpu.sync_copy(vals_v, acc.at[idx_v], add=True)

        plsc.subcore_barrier()
        pltpu.sync_copy(acc.at[pl.ds(s * rps, rps)],
                        out_hbm.at[pl.ds(s * rps, rps), pl.ds(c * Dh, Dh)])

    return sk(vals, idx, init)


def kernel(x, edge_index, edge_attr, batch, params):
    src = edge_index[0]
    dst = edge_index[1]
    ea8 = jnp.pad(edge_attr, ((0, 0), (0, 5)))
    xp = jnp.pad(x, ((0, 0), (0, 128 - x.shape[1])))
    pp = dict(params)
    pp['Wa_p'] = jnp.pad(params['Wa'], ((0, 128 - params['Wa'].shape[0]), (0, 0)))
    batch3 = batch.astype(F32).reshape(N // BN, 1, BN)
    znp = jnp.zeros((NP, 2 * 128), F32)
    E2 = E // 2
    HOFF = E2 // BE
    dst_a, dst_b = dst[:E2], dst[E2:]
    src_a, src_b = src[:E2], src[E2:]

    e = _compute_e(ea8, params['Wr'], params['br'])
    h0, td0, ts0 = _prep0(xp, pp)

    gd0a = _sc_gather(td0, dst_a)
    gs0a = _sc_gather(ts0, src_a)
    gd0b = _sc_gather(td0, dst_b)
    gs0b = _sc_gather(ts0, src_b)
    m0a = _edge_conv(gd0a, gs0a, e, params['l0'], 0)
    m0b = _edge_conv(gd0b, gs0b, e, params['l0'], HOFF)
    s0a = _sc_scatter_add(m0a, dst_a, znp, C)
    agg0 = _sc_scatter_add(m0b, dst_b, s0a, C)[:N]

    h1, hn1 = _update0(h0, agg0, params)
    gna = _sc_gather(hn1, src_a)
    gnb = _sc_gather(hn1, src_b)
    ma, mva = _edge_equi(gna, e, ea8, params, 0)
    mb, mvb = _edge_equi(gnb, e, ea8, params, HOFF)
    sea = _sc_scatter_add(ma, dst_a, znp, C)
    agg_eq = _sc_scatter_add(mb, dst_b, sea, C)[:N]
    va = _sc_scatter_add_esplit(mva, dst_a, znp)
    aggv = _sc_scatter_add_esplit(mvb, dst_b, va)[:N]

    h2, td2, ts2 = _update1(h1, agg_eq, aggv, params)
    gd2a = _sc_gather(td2, dst_a)
    gs2a = _sc_gather(ts2, src_a)
    gd2b = _sc_gather(td2, dst_b)
    gs2b = _sc_gather(ts2, src_b)
    m2a = _edge_conv(gd2a, gs2a, e, params['l2'], 0)
    m2b = _edge_conv(gd2b, gs2b, e, params['l2'], HOFF)
    s2a = _sc_scatter_add(m2a, dst_a, znp, C)
    agg2 = _sc_scatter_add(m2b, dst_b, s2a, C)[:N]

    out = _final(h2, agg2, batch3, params)
    return out.reshape(NG)
